# baseline (speedup 1.0000x reference)
import functools

import jax
import jax.numpy as jnp
from jax.experimental import pallas as pl
from jax.experimental.pallas import tpu as pltpu

F32 = jnp.float32


def _dot_tb(a, b):
    return jax.lax.dot_general(a, b, (((1,), (1,)), ((), ())),
                               preferred_element_type=F32)


def _encoder_kernel(nbatch, x_ref, w1_ref, b1_ref, w2_ref, b2_ref, w3_ref,
                    b3_ref, wfc_ref, bfc_ref, feat_ref):
    b1 = jnp.transpose(b1_ref[...])
    h = jnp.concatenate(
        [jax.lax.dot_general(x_ref[b], w1_ref[...], (((0,), (1,)), ((), ())),
                             preferred_element_type=F32)
         for b in range(nbatch)], axis=0)
    h = jnp.maximum(h + b1, 0.0)
    h = jnp.maximum(_dot_tb(h, w2_ref[...]) + jnp.transpose(b2_ref[...]), 0.0)
    h = _dot_tb(h, w3_ref[...]) + jnp.transpose(b3_ref[...])
    n = h.shape[0] // nbatch
    g = jnp.concatenate(
        [jnp.max(h[b * n:(b + 1) * n], axis=0, keepdims=True)
         for b in range(nbatch)], axis=0)
    feat = jnp.maximum(
        _dot_tb(g, wfc_ref[...]) + jnp.transpose(bfc_ref[...]), 0.0)
    feat_ref[...] = jnp.concatenate([feat, feat], axis=0)


def _decoder_kernel(vert_ref, w1v_ref, feat_ref, w1f_ref, b1_ref, w2_ref,
                    b2_ref, w3_ref, b3_ref, w4_ref, b4_ref, out_ref, vb_ref):
    b = pl.program_id(1)

    @pl.when(b == 0)
    def _():
        vb_ref[...] = _dot_tb(w1v_ref[0], vert_ref[...])

    featb = feat_ref[pl.ds(b, 1), :]
    d1row = _dot_tb(featb, w1f_ref[0])
    d1col = jnp.transpose(d1row)
    h = jnp.maximum(vb_ref[...] + d1col + b1_ref[0], 0.0)
    h = jnp.maximum(
        jnp.dot(w2_ref[0], h, preferred_element_type=F32) + b2_ref[0], 0.0)
    h = jnp.maximum(
        jnp.dot(w3_ref[0], h, preferred_element_type=F32) + b3_ref[0], 0.0)
    o = jnp.dot(w4_ref[0], h, preferred_element_type=F32) + b4_ref[0]
    out_ref[0, 0] = jnp.transpose(2.0 * jnp.tanh(o))


def kernel(x, enc_w1, enc_b1, enc_w2, enc_b2, enc_w3, enc_b3, fc_w, fc_b,
           dec_w1v, dec_w1f, dec_b1, dec_w2, dec_b2, dec_w3, dec_b3,
           dec_w4, dec_b4, vertex):
    B, _, N = x.shape
    P, C, _ = dec_w1v.shape
    V = vertex.shape[0]
    BN = fc_w.shape[0]

    feat = pl.pallas_call(
        functools.partial(_encoder_kernel, B),
        out_shape=jax.ShapeDtypeStruct((2 * B, BN), F32),
    )(x, enc_w1, enc_b1, enc_w2, enc_b2, enc_w3, enc_b3, fc_w, fc_b)

    w1f3 = dec_w1f.reshape(P, C, BN)
    b13 = dec_b1.reshape(P, C, 1)

    out4 = pl.pallas_call(
        _decoder_kernel,
        out_shape=jax.ShapeDtypeStruct((B, P, V, 3), F32),
        grid=(P, B),
        in_specs=[
            pl.BlockSpec((V, 3), lambda p, b: (0, 0)),
            pl.BlockSpec((1, C, 3), lambda p, b: (p, 0, 0)),
            pl.BlockSpec((2 * B, BN), lambda p, b: (0, 0)),
            pl.BlockSpec((1, C, BN), lambda p, b: (p, 0, 0)),
            pl.BlockSpec((1, C, 1), lambda p, b: (p, 0, 0)),
            pl.BlockSpec((1,) + dec_w2.shape[1:], lambda p, b: (p, 0, 0)),
            pl.BlockSpec((1,) + dec_b2.shape[1:], lambda p, b: (p, 0, 0)),
            pl.BlockSpec((1,) + dec_w3.shape[1:], lambda p, b: (p, 0, 0)),
            pl.BlockSpec((1,) + dec_b3.shape[1:], lambda p, b: (p, 0, 0)),
            pl.BlockSpec((1,) + dec_w4.shape[1:], lambda p, b: (p, 0, 0)),
            pl.BlockSpec((1,) + dec_b4.shape[1:], lambda p, b: (p, 0, 0)),
        ],
        out_specs=pl.BlockSpec((1, 1, V, 3), lambda p, b: (b, p, 0, 0)),
        scratch_shapes=[pltpu.VMEM((C, V), F32)],
        compiler_params=pltpu.CompilerParams(
            dimension_semantics=("arbitrary", "arbitrary")),
    )(vertex, dec_w1v, feat, w1f3, b13, dec_w2, dec_b2, dec_w3, dec_b3,
      dec_w4, dec_b4)

    return out4.reshape(B, P * V, 3)


# speedup vs baseline: 1.1864x; 1.1864x over previous
import functools

import jax
import jax.numpy as jnp
from jax.experimental import pallas as pl
from jax.experimental.pallas import tpu as pltpu

F32 = jnp.float32


def _dot_tb(a, b):
    return jax.lax.dot_general(a, b, (((1,), (1,)), ((), ())),
                               preferred_element_type=F32)


def _encoder_kernel(nbatch, nprim, x_ref, w1_ref, w2_ref, w3_ref, wfc_ref,
                    w1f_ref, bias_ref, d1_ref):
    c1 = w1_ref.shape[0]
    c2 = w2_ref.shape[0]
    c3 = w3_ref.shape[0]
    o1, o2, o3 = c1, c1 + c2, c1 + c2 + c3
    b1 = jnp.transpose(bias_ref[0:o1])
    b2 = jnp.transpose(bias_ref[o1:o2])
    b3 = jnp.transpose(bias_ref[o2:o3])
    bfc = jnp.transpose(bias_ref[o3:])
    h = jnp.concatenate(
        [jax.lax.dot_general(x_ref[b], w1_ref[...], (((0,), (1,)), ((), ())),
                             preferred_element_type=F32)
         for b in range(nbatch)], axis=0)
    h = jnp.maximum(h + b1, 0.0)
    h = jnp.maximum(_dot_tb(h, w2_ref[...]) + b2, 0.0)
    h = _dot_tb(h, w3_ref[...]) + b3
    n = h.shape[0] // nbatch
    g = jnp.concatenate(
        [jnp.max(h[b * n:(b + 1) * n], axis=0, keepdims=True)
         for b in range(nbatch)], axis=0)
    feat = jnp.maximum(_dot_tb(g, wfc_ref[...]) + bfc, 0.0)
    feat = jnp.concatenate([feat, feat], axis=0)
    d1 = jnp.transpose(_dot_tb(feat, w1f_ref[...]))
    C = d1.shape[0] // nprim
    for p in range(nprim):
        d1_ref[p] = d1[p * C:(p + 1) * C]


def _decoder_kernel(nbatch, coff, vert_ref, w1v_ref, d1_ref, bias_ref,
                    w2_ref, w3_ref, w4_ref, out_ref):
    c1, c2, c3, c4 = coff
    C = d1_ref.shape[1]
    b1 = bias_ref[0, c1:c1 + C]
    b2 = bias_ref[0, c2:c2 + w2_ref.shape[1]]
    b3 = bias_ref[0, c3:c3 + w3_ref.shape[1]]
    b4 = bias_ref[0, c4:c4 + w4_ref.shape[1]]
    vb = _dot_tb(w1v_ref[0], vert_ref[...]) + b1
    for b in range(nbatch):
        h = jnp.maximum(vb + d1_ref[0, :, b:b + 1], 0.0)
        h = jnp.maximum(
            jnp.dot(w2_ref[0], h, preferred_element_type=F32) + b2, 0.0)
        h = jnp.maximum(
            jnp.dot(w3_ref[0], h, preferred_element_type=F32) + b3, 0.0)
        o = jnp.dot(w4_ref[0], h, preferred_element_type=F32) + b4
        out_ref[b, 0] = 2.0 * jnp.tanh(o)


def _pad8(n):
    return (-n) % 8


def kernel(x, enc_w1, enc_b1, enc_w2, enc_b2, enc_w3, enc_b3, fc_w, fc_b,
           dec_w1v, dec_w1f, dec_b1, dec_w2, dec_b2, dec_w3, dec_b3,
           dec_w4, dec_b4, vertex):
    B, _, N = x.shape
    P, C, _ = dec_w1v.shape
    V = vertex.shape[0]
    BN = fc_w.shape[0]
    C2 = dec_w2.shape[1]
    C3 = dec_w3.shape[1]

    ebias = jnp.concatenate([enc_b1, enc_b2, enc_b3, fc_b], axis=0)

    d1cols = pl.pallas_call(
        functools.partial(_encoder_kernel, B, P),
        out_shape=jax.ShapeDtypeStruct((P, C, 2 * B), F32),
    )(x, enc_w1, enc_w2, enc_w3, fc_w, dec_w1f, ebias)

    z = lambda k: jnp.zeros((P, k, 1), F32)
    p1, p2 = _pad8(C), _pad8(C2)
    dbias = jnp.concatenate(
        [dec_b1.reshape(P, C, 1), z(p1), dec_b2, z(p2), dec_b3, dec_b4],
        axis=1)
    coff = (0, C + p1, C + p1 + C2 + p2, C + p1 + C2 + p2 + C3)

    out4 = pl.pallas_call(
        functools.partial(_decoder_kernel, B, coff),
        out_shape=jax.ShapeDtypeStruct((B, P, 3, V), F32),
        grid=(P,),
        in_specs=[
            pl.BlockSpec((V, 3), lambda p: (0, 0)),
            pl.BlockSpec((1, C, 3), lambda p: (p, 0, 0)),
            pl.BlockSpec((1, C, 2 * B), lambda p: (p, 0, 0)),
            pl.BlockSpec((1,) + dbias.shape[1:], lambda p: (p, 0, 0)),
            pl.BlockSpec((1,) + dec_w2.shape[1:], lambda p: (p, 0, 0)),
            pl.BlockSpec((1,) + dec_w3.shape[1:], lambda p: (p, 0, 0)),
            pl.BlockSpec((1,) + dec_w4.shape[1:], lambda p: (p, 0, 0)),
        ],
        out_specs=pl.BlockSpec((B, 1, 3, V), lambda p: (0, p, 0, 0)),
        compiler_params=pltpu.CompilerParams(
            dimension_semantics=("arbitrary",)),
    )(vertex, dec_w1v, d1cols, dbias, dec_w2, dec_w3, dec_w4)

    return jnp.transpose(out4, (0, 1, 3, 2)).reshape(B, P * V, 3)


# speedup vs baseline: 1.2723x; 1.0724x over previous
import functools

import jax
import jax.numpy as jnp
from jax.experimental import pallas as pl
from jax.experimental.pallas import tpu as pltpu

F32 = jnp.float32
BF16 = jnp.bfloat16


def _dot_tb(a, b):
    return jax.lax.dot_general(a, b, (((1,), (1,)), ((), ())),
                               preferred_element_type=F32)


def _encoder_kernel(nbatch, nprim, x_ref, w1_ref, b1_ref, w2_ref, b2_ref,
                    w3_ref, b3_ref, wfc_hbm, bfc_ref, w1f_hbm, db1_ref,
                    d1_ref, wfc_s, w1f_s, sem_fc, sem_1f):
    cp_fc = pltpu.make_async_copy(wfc_hbm, wfc_s, sem_fc)
    cp_fc.start()
    cp_1f = pltpu.make_async_copy(w1f_hbm, w1f_s, sem_1f)
    cp_1f.start()
    h = jnp.concatenate(
        [jax.lax.dot_general(x_ref[b], w1_ref[...], (((0,), (1,)), ((), ())),
                             preferred_element_type=F32)
         for b in range(nbatch)], axis=0)
    h = jnp.maximum(h + jnp.transpose(b1_ref[...]), 0.0)
    h = jnp.maximum(_dot_tb(h, w2_ref[...]) + jnp.transpose(b2_ref[...]),
                    0.0)
    h = _dot_tb(h, w3_ref[...]) + jnp.transpose(b3_ref[...])
    n = h.shape[0] // nbatch
    g = jnp.concatenate(
        [jnp.max(h[b * n:(b + 1) * n], axis=0, keepdims=True)
         for b in range(nbatch)], axis=0)
    cp_fc.wait()
    feat = jnp.maximum(
        _dot_tb(g, wfc_s[...]) + jnp.transpose(bfc_ref[...]), 0.0)
    feat = jnp.concatenate([feat, feat], axis=0)
    cp_1f.wait()
    d1 = jnp.transpose(_dot_tb(feat, w1f_s[...]))
    d1 = d1 + db1_ref[...]
    C = d1.shape[0] // nprim
    for p in range(nprim):
        d1_ref[p] = d1[p * C:(p + 1) * C]


def _decoder_kernel(nbatch, vert_ref, w1v_ref, d1_ref, b2_ref, b3_ref,
                    b4_ref, w2_ref, w3_ref, w4_ref, out_ref):
    vb = _dot_tb(w1v_ref[0], vert_ref[...]).astype(BF16)
    w2 = w2_ref[0].astype(BF16)
    w3 = w3_ref[0].astype(BF16)
    w4 = w4_ref[0].astype(BF16)
    d1 = d1_ref[0].astype(BF16)
    b2 = b2_ref[0]
    b3 = b3_ref[0]
    b4 = b4_ref[0]
    zero = jnp.array(0.0, BF16)
    for b in range(nbatch):
        h = jnp.maximum(vb + d1[:, b:b + 1], zero)
        h = jnp.maximum(
            jnp.dot(w2, h, preferred_element_type=F32) + b2, 0.0
        ).astype(BF16)
        h = jnp.maximum(
            jnp.dot(w3, h, preferred_element_type=F32) + b3, 0.0
        ).astype(BF16)
        o = jnp.dot(w4, h, preferred_element_type=F32) + b4
        out_ref[b, 0] = 2.0 * jnp.tanh(o)


def kernel(x, enc_w1, enc_b1, enc_w2, enc_b2, enc_w3, enc_b3, fc_w, fc_b,
           dec_w1v, dec_w1f, dec_b1, dec_w2, dec_b2, dec_w3, dec_b3,
           dec_w4, dec_b4, vertex):
    B, _, N = x.shape
    P, C, _ = dec_w1v.shape
    V = vertex.shape[0]
    BN = fc_w.shape[0]

    d1cols = pl.pallas_call(
        functools.partial(_encoder_kernel, B, P),
        out_shape=jax.ShapeDtypeStruct((P, C, 2 * B), F32),
        in_specs=[pl.BlockSpec(memory_space=pl.ANY) if i in (7, 9)
                  else pl.BlockSpec(memory_space=pltpu.MemorySpace.VMEM)
                  for i in range(11)],
        scratch_shapes=[pltpu.VMEM(fc_w.shape, F32),
                        pltpu.VMEM(dec_w1f.shape, F32),
                        pltpu.SemaphoreType.DMA,
                        pltpu.SemaphoreType.DMA],
    )(x, enc_w1, enc_b1, enc_w2, enc_b2, enc_w3, enc_b3, fc_w, fc_b,
      dec_w1f, dec_b1)

    out4 = pl.pallas_call(
        functools.partial(_decoder_kernel, B),
        out_shape=jax.ShapeDtypeStruct((B, P, 3, V), F32),
        grid=(P,),
        in_specs=[
            pl.BlockSpec((V, 3), lambda p: (0, 0)),
            pl.BlockSpec((1, C, 3), lambda p: (p, 0, 0)),
            pl.BlockSpec((1, C, 2 * B), lambda p: (p, 0, 0)),
            pl.BlockSpec((1,) + dec_b2.shape[1:], lambda p: (p, 0, 0)),
            pl.BlockSpec((1,) + dec_b3.shape[1:], lambda p: (p, 0, 0)),
            pl.BlockSpec((1,) + dec_b4.shape[1:], lambda p: (p, 0, 0)),
            pl.BlockSpec((1,) + dec_w2.shape[1:], lambda p: (p, 0, 0)),
            pl.BlockSpec((1,) + dec_w3.shape[1:], lambda p: (p, 0, 0)),
            pl.BlockSpec((1,) + dec_w4.shape[1:], lambda p: (p, 0, 0)),
        ],
        out_specs=pl.BlockSpec((B, 1, 3, V), lambda p: (0, p, 0, 0)),
        compiler_params=pltpu.CompilerParams(
            dimension_semantics=("arbitrary",)),
    )(vertex, dec_w1v, d1cols, dec_b2, dec_b3, dec_b4, dec_w2, dec_w3,
      dec_w4)

    return jnp.transpose(out4, (0, 1, 3, 2)).reshape(B, P * V, 3)


# speedup vs baseline: 1.3331x; 1.0478x over previous
import functools

import jax
import jax.numpy as jnp
from jax.experimental import pallas as pl
from jax.experimental.pallas import tpu as pltpu

F32 = jnp.float32
BF16 = jnp.bfloat16


def _dot_tb(a, b):
    return jax.lax.dot_general(a, b, (((1,), (1,)), ((), ())),
                               preferred_element_type=F32)


def _encoder_kernel(nbatch, nprim, x_ref, w1_ref, w2_ref, w3_ref, wfc_hbm,
                    w1f_hbm, bias_ref, d1_ref, wfc_s, w1f_s, sem_fc,
                    sem_1f):
    cp_fc = pltpu.make_async_copy(wfc_hbm, wfc_s, sem_fc)
    cp_fc.start()
    cp_1f = pltpu.make_async_copy(w1f_hbm, w1f_s, sem_1f)
    cp_1f.start()
    c1 = w1_ref.shape[0]
    c2 = w2_ref.shape[0]
    c3 = w3_ref.shape[0]
    bn = wfc_s.shape[0]
    o1, o2, o3, o4 = c1, c1 + c2, c1 + c2 + c3, c1 + c2 + c3 + bn
    h = jnp.concatenate(
        [jax.lax.dot_general(x_ref[b], w1_ref[...], (((0,), (1,)), ((), ())),
                             preferred_element_type=F32)
         for b in range(nbatch)], axis=0)
    h = jnp.maximum(h + jnp.transpose(bias_ref[0:o1]), 0.0)
    h = jnp.maximum(_dot_tb(h, w2_ref[...]) + jnp.transpose(bias_ref[o1:o2]),
                    0.0)
    h = _dot_tb(h, w3_ref[...]) + jnp.transpose(bias_ref[o2:o3])
    n = h.shape[0] // nbatch
    g = jnp.concatenate(
        [jnp.max(h[b * n:(b + 1) * n], axis=0, keepdims=True)
         for b in range(nbatch)], axis=0)
    cp_fc.wait()
    feat = jnp.maximum(
        _dot_tb(g, wfc_s[...]) + jnp.transpose(bias_ref[o3:o4]), 0.0)
    feat = jnp.concatenate([feat, feat], axis=0)
    cp_1f.wait()
    d1 = jnp.transpose(_dot_tb(feat, w1f_s[...]))
    d1 = d1 + bias_ref[o4:]
    C = d1.shape[0] // nprim
    for p in range(nprim):
        d1_ref[p] = d1[p * C:(p + 1) * C]


def _decoder_kernel(nbatch, vert_ref, w1v_ref, d1_ref, b2_ref, b3_ref,
                    b4_ref, w2_ref, w3_ref, w4_ref, out_ref):
    vb = _dot_tb(w1v_ref[0], vert_ref[...]).astype(BF16)
    w2 = w2_ref[0].astype(BF16)
    w3 = w3_ref[0].astype(BF16)
    w4 = w4_ref[0].astype(BF16)
    d1 = d1_ref[0].astype(BF16)
    b2 = b2_ref[0]
    b3 = b3_ref[0]
    b4 = b4_ref[0]
    zero = jnp.array(0.0, BF16)
    for b in range(nbatch):
        h = jnp.maximum(vb + d1[:, b:b + 1], zero)
        h = jnp.maximum(
            jnp.dot(w2, h, preferred_element_type=F32) + b2, 0.0
        ).astype(BF16)
        h = jnp.maximum(
            jnp.dot(w3, h, preferred_element_type=F32) + b3, 0.0
        ).astype(BF16)
        o = jnp.dot(w4, h, preferred_element_type=F32) + b4
        out_ref[b, 0] = 2.0 * jnp.tanh(o)


def kernel(x, enc_w1, enc_b1, enc_w2, enc_b2, enc_w3, enc_b3, fc_w, fc_b,
           dec_w1v, dec_w1f, dec_b1, dec_w2, dec_b2, dec_w3, dec_b3,
           dec_w4, dec_b4, vertex):
    B, _, N = x.shape
    P, C, _ = dec_w1v.shape
    V = vertex.shape[0]
    BN = fc_w.shape[0]

    ebias = jnp.concatenate([enc_b1, enc_b2, enc_b3, fc_b, dec_b1], axis=0)

    d1cols = pl.pallas_call(
        functools.partial(_encoder_kernel, B, P),
        out_shape=jax.ShapeDtypeStruct((P, C, 2 * B), F32),
        in_specs=[pl.BlockSpec(memory_space=pl.ANY) if i in (4, 5)
                  else pl.BlockSpec(memory_space=pltpu.MemorySpace.VMEM)
                  for i in range(7)],
        scratch_shapes=[pltpu.VMEM(fc_w.shape, F32),
                        pltpu.VMEM(dec_w1f.shape, F32),
                        pltpu.SemaphoreType.DMA,
                        pltpu.SemaphoreType.DMA],
    )(x, enc_w1, enc_w2, enc_w3, fc_w, dec_w1f, ebias)

    out4 = pl.pallas_call(
        functools.partial(_decoder_kernel, B),
        out_shape=jax.ShapeDtypeStruct((B, P, 3, V), F32),
        grid=(P,),
        in_specs=[
            pl.BlockSpec((V, 3), lambda p: (0, 0)),
            pl.BlockSpec((1, C, 3), lambda p: (p, 0, 0)),
            pl.BlockSpec((1, C, 2 * B), lambda p: (p, 0, 0)),
            pl.BlockSpec((1,) + dec_b2.shape[1:], lambda p: (p, 0, 0)),
            pl.BlockSpec((1,) + dec_b3.shape[1:], lambda p: (p, 0, 0)),
            pl.BlockSpec((1,) + dec_b4.shape[1:], lambda p: (p, 0, 0)),
            pl.BlockSpec((1,) + dec_w2.shape[1:], lambda p: (p, 0, 0)),
            pl.BlockSpec((1,) + dec_w3.shape[1:], lambda p: (p, 0, 0)),
            pl.BlockSpec((1,) + dec_w4.shape[1:], lambda p: (p, 0, 0)),
        ],
        out_specs=pl.BlockSpec((B, 1, 3, V), lambda p: (0, p, 0, 0)),
        compiler_params=pltpu.CompilerParams(
            dimension_semantics=("arbitrary",)),
    )(vertex, dec_w1v, d1cols, dec_b2, dec_b3, dec_b4, dec_w2, dec_w3,
      dec_w4)

    return jnp.transpose(out4, (0, 1, 3, 2)).reshape(B, P * V, 3)
